# baseline (speedup 1.0000x reference)
import jax
import jax.numpy as jnp
from jax.experimental import pallas as pl
from jax.experimental.pallas import tpu as pltpu

_B, _P, _D = 64, 1024, 768
_PB = 256


def _body(x_ref, t_ref, o_ref):
    o_ref[...] = x_ref[...] + t_ref[...]


def kernel(x, table):
    return pl.pallas_call(
        _body,
        grid=(_P // _PB, _B),
        in_specs=[
            pl.BlockSpec((1, _PB, _D), lambda p, b: (b, p, 0)),
            pl.BlockSpec((_PB, _D), lambda p, b: (p, 0)),
        ],
        out_specs=pl.BlockSpec((1, _PB, _D), lambda p, b: (b, p, 0)),
        out_shape=jax.ShapeDtypeStruct((_B, _P, _D), jnp.float32),
        compiler_params=pltpu.CompilerParams(
            dimension_semantics=("arbitrary", "arbitrary"),
        ),
    )(x, table)


# speedup vs baseline: 1.7368x; 1.7368x over previous
import jax
import jax.numpy as jnp
from jax.experimental import pallas as pl
from jax.experimental.pallas import tpu as pltpu

_B, _P, _D = 64, 1024, 768
_PB = 256


def _body(x_ref, t_ref, o_ref):
    o_ref[...] = x_ref[...] + t_ref[...]


def kernel(x, table):
    return pl.pallas_call(
        _body,
        grid=(_B,),
        in_specs=[
            pl.BlockSpec((1, _P, _D), lambda b: (b, 0, 0)),
            pl.BlockSpec((_P, _D), lambda b: (0, 0)),
        ],
        out_specs=pl.BlockSpec((1, _P, _D), lambda b: (b, 0, 0)),
        out_shape=jax.ShapeDtypeStruct((_B, _P, _D), jnp.float32),
        compiler_params=pltpu.CompilerParams(
            dimension_semantics=("arbitrary",),
        ),
    )(x, table)


# speedup vs baseline: 1.7903x; 1.0308x over previous
import jax
import jax.numpy as jnp
from jax.experimental import pallas as pl
from jax.experimental.pallas import tpu as pltpu

_B, _P, _D = 64, 1024, 768
_PB = 256


def _body(x_ref, t_ref, o_ref):
    o_ref[...] = x_ref[...] + t_ref[...]


def kernel(x, table):
    return pl.pallas_call(
        _body,
        grid=(_B // 2,),
        in_specs=[
            pl.BlockSpec((2, _P, _D), lambda b: (b, 0, 0)),
            pl.BlockSpec((_P, _D), lambda b: (0, 0)),
        ],
        out_specs=pl.BlockSpec((2, _P, _D), lambda b: (b, 0, 0)),
        out_shape=jax.ShapeDtypeStruct((_B, _P, _D), jnp.float32),
        compiler_params=pltpu.CompilerParams(
            dimension_semantics=("arbitrary",),
        ),
    )(x, table)


# speedup vs baseline: 1.8094x; 1.0106x over previous
import jax
import jax.numpy as jnp
from jax.experimental import pallas as pl
from jax.experimental.pallas import tpu as pltpu

_B, _P, _D = 64, 1024, 768
_PB = 256


def _body(x_ref, t_ref, o_ref):
    o_ref[...] = x_ref[...] + t_ref[...]


def kernel(x, table):
    return pl.pallas_call(
        _body,
        grid=(_B // 4,),
        in_specs=[
            pl.BlockSpec((4, _P, _D), lambda b: (b, 0, 0)),
            pl.BlockSpec((_P, _D), lambda b: (0, 0)),
        ],
        out_specs=pl.BlockSpec((4, _P, _D), lambda b: (b, 0, 0)),
        out_shape=jax.ShapeDtypeStruct((_B, _P, _D), jnp.float32),
        compiler_params=pltpu.CompilerParams(
            dimension_semantics=("arbitrary",),
        ),
    )(x, table)
